# baseline (speedup 1.0000x reference)
import functools

import jax
import jax.numpy as jnp
from jax import lax
from jax.experimental import pallas as pl
from jax.experimental.pallas import tpu as pltpu
from jax.experimental.pallas import tpu_sc as plsc

N = 10000
IN_CH = 128
HID = 16
OUT_CH = 128

NC = 2
NS = 16
NW = NC * NS
CHUNK = 128
ECH = 2500
NCH = ECH // NW
XCH = ECH - NCH * NW
NBUF = 6
NG = NCH // NBUF
NPAD = 10240
RPT = NPAD // NS
NF = NPAD * HID // 128
NFV = N * HID // 128


def _sc_mesh():
    return plsc.VectorSubcoreMesh(core_axis_name="c", subcore_axis_name="s")


def _sc_degree(ei_raw):
    @functools.partial(
        pl.kernel,
        out_type=jax.ShapeDtypeStruct((NC, NPAD, HID), jnp.float32),
        mesh=_sc_mesh(),
        compiler_params=pltpu.CompilerParams(use_tc_tiling_on_sc=False),
        scratch_types=[
            pltpu.VMEM((NCH * CHUNK,), jnp.int32),
            pltpu.VMEM((NCH, CHUNK), jnp.int32),
            pltpu.VMEM((XCH * CHUNK,), jnp.int32),
            pltpu.VMEM((XCH, CHUNK), jnp.int32),
            pltpu.VMEM((CHUNK,), jnp.float32),
            pltpu.VMEM((RPT,), jnp.float32),
            pltpu.VMEM((RPT, HID), jnp.float32),
            pltpu.VMEM_SHARED((NPAD,), jnp.float32),
        ] + [pltpu.SemaphoreType.DMA] * NBUF,
    )
    def k(ei_hbm, out_hbm, didx1, didx, dxidx1, dxidx, ones_v,
          dv, rep, acc, *ssem):
        cid = lax.axis_index("c")
        sid = lax.axis_index("s")
        wid = cid * NS + sid
        last = wid == NW - 1
        r0 = sid * RPT
        pltpu.sync_copy(ei_hbm.at[1, pl.ds(wid * NCH * CHUNK, NCH * CHUNK)],
                        didx1)

        @pl.when(last)
        def _():
            pltpu.sync_copy(ei_hbm.at[1, pl.ds(NW * NCH * CHUNK,
                                               XCH * CHUNK)], dxidx1)

        for m in range(CHUNK // HID):
            ones_v[pl.ds(m * HID, HID)] = jnp.ones((HID,), jnp.float32)

        def zbody(i, carry):
            dv[pl.ds(i * HID, HID)] = jnp.zeros((HID,), jnp.float32)
            return carry

        lax.fori_loop(0, RPT // HID, zbody, 0)
        pltpu.sync_copy(dv, acc.at[pl.ds(r0, RPT)])

        def rbody(j, carry):
            for m in range(CHUNK // HID):
                didx[j, pl.ds(m * HID, HID)] = (
                    didx1[pl.ds(j * CHUNK + m * HID, HID)])
            return carry

        lax.fori_loop(0, NCH, rbody, 0)

        @pl.when(last)
        def _():
            def xbody(j, carry):
                for m in range(CHUNK // HID):
                    dxidx[j, pl.ds(m * HID, HID)] = (
                        dxidx1[pl.ds(j * CHUNK + m * HID, HID)])
                return carry

            lax.fori_loop(0, XCH, xbody, 0)

        plsc.subcore_barrier()

        def body(g, carry):
            for b in range(NBUF):
                pltpu.async_copy(ones_v, acc.at[didx.at[g * NBUF + b]],
                                 ssem[b], add=True)
            for b in range(NBUF):
                pltpu.make_async_copy(
                    ones_v, acc.at[didx.at[g * NBUF + b]], ssem[b]).wait()
            return carry

        lax.fori_loop(0, NG, body, 0)

        @pl.when(last)
        def _():
            for t in range(XCH):
                pltpu.sync_copy(ones_v, acc.at[dxidx.at[t]], add=True)

        plsc.subcore_barrier()
        pltpu.sync_copy(acc.at[pl.ds(r0, RPT)], dv)

        def repl(g, carry):
            v16 = dv[pl.ds(g * HID, HID)]
            for t in range(HID):
                rep[g * HID + t] = jnp.full((HID,), v16[t], jnp.float32)
            return carry

        lax.fori_loop(0, RPT // HID, repl, 0)
        pltpu.sync_copy(rep, out_hbm.at[cid, pl.ds(r0, RPT)])

    return k(ei_raw)


def _sc_aggregate(table, ei3):
    @functools.partial(
        pl.kernel,
        out_type=jax.ShapeDtypeStruct((NC, NPAD, HID), jnp.float32),
        mesh=_sc_mesh(),
        compiler_params=pltpu.CompilerParams(use_tc_tiling_on_sc=False),
        scratch_types=[
            pltpu.VMEM((NCH, CHUNK), jnp.int32),
            pltpu.VMEM((NCH, CHUNK), jnp.int32),
            pltpu.VMEM((XCH, CHUNK), jnp.int32),
            pltpu.VMEM((XCH, CHUNK), jnp.int32),
            pltpu.VMEM((RPT, HID), jnp.float32),
            pltpu.VMEM_SHARED((NPAD, HID), jnp.float32),
            pltpu.VMEM_SHARED((NPAD, HID), jnp.float32),
        ]
        + [pltpu.VMEM((CHUNK, HID), jnp.float32)] * NBUF
        + [pltpu.SemaphoreType.DMA] * (2 * NBUF),
    )
    def k(table_hbm, ei_hbm, out_hbm,
          sidx, didx, sxidx, dxidx, zbuf, acc, tsp, *rest):
        bufs = rest[:NBUF]
        gsem = rest[NBUF:2 * NBUF]
        ssem = rest[2 * NBUF:]
        cid = lax.axis_index("c")
        sid = lax.axis_index("s")
        wid = cid * NS + sid
        last = wid == NW - 1
        r0 = sid * RPT

        def zbody(i, carry):
            zbuf[i] = jnp.zeros((HID,), jnp.float32)
            return carry

        lax.fori_loop(0, RPT, zbody, 0)
        pltpu.sync_copy(zbuf, acc.at[pl.ds(r0, RPT)])
        pltpu.sync_copy(table_hbm.at[pl.ds(r0, RPT)], tsp.at[pl.ds(r0, RPT)])
        pltpu.sync_copy(ei_hbm.at[0, pl.ds(wid * NCH, NCH)], sidx)
        pltpu.sync_copy(ei_hbm.at[1, pl.ds(wid * NCH, NCH)], didx)

        @pl.when(last)
        def _():
            pltpu.sync_copy(ei_hbm.at[0, pl.ds(NW * NCH, XCH)], sxidx)
            pltpu.sync_copy(ei_hbm.at[1, pl.ds(NW * NCH, XCH)], dxidx)

        plsc.subcore_barrier()

        for b in range(NBUF):
            pltpu.async_copy(tsp.at[sidx.at[b]], bufs[b], gsem[b])

        def body(g, carry):
            for b in range(NBUF):
                j = g * NBUF + b
                pltpu.make_async_copy(tsp.at[sidx.at[j]], bufs[b],
                                      gsem[b]).wait()
                pltpu.async_copy(bufs[b], acc.at[didx.at[j]], ssem[b],
                                 add=True)
            for b in range(NBUF):
                j = g * NBUF + b
                pltpu.make_async_copy(bufs[b], acc.at[didx.at[j]],
                                      ssem[b]).wait()

                @pl.when(g + 1 < NG)
                def _():
                    pltpu.async_copy(tsp.at[sidx.at[j + NBUF]], bufs[b],
                                     gsem[b])

            return carry

        lax.fori_loop(0, NG, body, 0)

        @pl.when(last)
        def _():
            for t in range(XCH):
                pltpu.async_copy(tsp.at[sxidx.at[t]], bufs[0],
                                 gsem[0]).wait()
                pltpu.sync_copy(bufs[0], acc.at[dxidx.at[t]], add=True)

        plsc.subcore_barrier()
        pltpu.sync_copy(acc.at[pl.ds(r0, RPT)],
                        out_hbm.at[cid, pl.ds(r0, RPT)])

    return k(table, ei3)


def _tc_matmul(xr, w1):
    def body(xr_ref, w1_ref, z1f_ref):
        parts = [
            jnp.dot(xr_ref[:, k, :], w1_ref[...],
                    preferred_element_type=jnp.float32)
            for k in range(8)
        ]
        z1f_ref[pl.ds(0, NFV)] = jnp.concatenate(parts, axis=1)
        z1f_ref[pl.ds(NFV, NF - NFV)] = jnp.zeros((NF - NFV, 128),
                                                  jnp.float32)

    return pl.pallas_call(
        body,
        out_shape=jax.ShapeDtypeStruct((NF, 128), jnp.float32),
    )(xr, w1)


def _tc_scale(z1f, degpf):
    def body(z1f_ref, degpf_ref, zt1f_ref, dinvf_ref):
        deg = degpf_ref[0] + degpf_ref[1] + 1.0
        dinv = lax.rsqrt(deg)
        zt1f_ref[...] = dinv * z1f_ref[...]
        dinvf_ref[...] = dinv

    return pl.pallas_call(
        body,
        out_shape=(
            jax.ShapeDtypeStruct((NF, 128), jnp.float32),
            jax.ShapeDtypeStruct((NF, 128), jnp.float32),
        ),
    )(z1f, degpf)


def _tc_mid(aggpf, zt1f, dinvf, b1f):
    def body(ap_ref, zt1f_ref, dinvf_ref, b1f_ref, zt2f_ref):
        dinv = dinvf_ref[...]
        pre = dinv * (ap_ref[0] + ap_ref[1] + zt1f_ref[...]) + b1f_ref[...]
        zt2f_ref[...] = dinv * jnp.maximum(pre, 0.0)

    return pl.pallas_call(
        body,
        out_shape=jax.ShapeDtypeStruct((NF, 128), jnp.float32),
    )(aggpf, zt1f, dinvf, b1f)


def _tc_post(aggpf, zt2f, dinvf, w2, b2):
    def body(ap_ref, zt2f_ref, dinvf_ref, w2_ref, b2_ref, out_ref):
        g = dinvf_ref[...] * (ap_ref[0] + ap_ref[1] + zt2f_ref[...])
        gv = g[:NFV]
        for k in range(8):
            out_ref[:, k, :] = jnp.dot(
                gv[:, 16 * k:16 * (k + 1)], w2_ref[...],
                preferred_element_type=jnp.float32) + b2_ref[...]

    return pl.pallas_call(
        body,
        out_shape=jax.ShapeDtypeStruct((NFV, 8, 128), jnp.float32),
    )(aggpf, zt2f, dinvf, w2, b2)


def kernel(x, edge_index, W1, b1, W2, b2):
    ei = edge_index.astype(jnp.int32)
    ei3 = ei.reshape(2, ECH, CHUNK)
    xr = x.reshape(NFV, 8, 128)

    b1f = jnp.tile(b1, 8).reshape(1, 128)

    degp = _sc_degree(ei)
    z1f = _tc_matmul(xr, W1)
    zt1f, dinvf = _tc_scale(z1f, degp.reshape(NC, NF, 128))
    agg1 = _sc_aggregate(zt1f.reshape(NPAD, HID), ei3)
    zt2f = _tc_mid(agg1.reshape(NC, NF, 128), zt1f, dinvf, b1f)
    agg2 = _sc_aggregate(zt2f.reshape(NPAD, HID), ei3)
    out3 = _tc_post(agg2.reshape(NC, NF, 128), zt2f, dinvf, W2,
                    b2.reshape(1, OUT_CH))
    return out3.reshape(N, OUT_CH)


# speedup vs baseline: 1.0417x; 1.0417x over previous
import functools

import jax
import jax.numpy as jnp
from jax import lax
from jax.experimental import pallas as pl
from jax.experimental.pallas import tpu as pltpu
from jax.experimental.pallas import tpu_sc as plsc

N = 10000
IN_CH = 128
HID = 16
OUT_CH = 128

NC = 2
NS = 16
NW = NC * NS
CHUNK = 128
ECH = 2500
NCH = ECH // NW
XCH = ECH - NCH * NW
NBUF = 6
NG = NCH // NBUF
NPAD = 10240
RPT = NPAD // NS
NF = NPAD * HID // 128
NFV = N * HID // 128


def _sc_mesh():
    return plsc.VectorSubcoreMesh(core_axis_name="c", subcore_axis_name="s")


def _sc_degree(ei_raw):
    @functools.partial(
        pl.kernel,
        out_type=jax.ShapeDtypeStruct((NC, NPAD, HID), jnp.float32),
        mesh=_sc_mesh(),
        compiler_params=pltpu.CompilerParams(use_tc_tiling_on_sc=False),
        scratch_types=[
            pltpu.VMEM((NCH * CHUNK,), jnp.int32),
            pltpu.VMEM((NCH, CHUNK), jnp.int32),
            pltpu.VMEM((XCH * CHUNK,), jnp.int32),
            pltpu.VMEM((XCH, CHUNK), jnp.int32),
            pltpu.VMEM((CHUNK,), jnp.float32),
            pltpu.VMEM((RPT,), jnp.float32),
            pltpu.VMEM((RPT, HID), jnp.float32),
            pltpu.VMEM_SHARED((NPAD,), jnp.float32),
        ] + [pltpu.SemaphoreType.DMA] * NBUF,
    )
    def k(ei_hbm, out_hbm, didx1, didx, dxidx1, dxidx, ones_v,
          dv, rep, acc, *ssem):
        cid = lax.axis_index("c")
        sid = lax.axis_index("s")
        wid = cid * NS + sid
        last = wid == NW - 1
        r0 = sid * RPT
        pltpu.sync_copy(ei_hbm.at[1, pl.ds(wid * NCH * CHUNK, NCH * CHUNK)],
                        didx1)

        @pl.when(last)
        def _():
            pltpu.sync_copy(ei_hbm.at[1, pl.ds(NW * NCH * CHUNK,
                                               XCH * CHUNK)], dxidx1)

        for m in range(CHUNK // HID):
            ones_v[pl.ds(m * HID, HID)] = jnp.ones((HID,), jnp.float32)

        def zbody(i, carry):
            dv[pl.ds(i * HID, HID)] = jnp.zeros((HID,), jnp.float32)
            return carry

        lax.fori_loop(0, RPT // HID, zbody, 0)
        pltpu.sync_copy(dv, acc.at[pl.ds(r0, RPT)])

        def rbody(j, carry):
            for m in range(CHUNK // HID):
                didx[j, pl.ds(m * HID, HID)] = (
                    didx1[pl.ds(j * CHUNK + m * HID, HID)])
            return carry

        lax.fori_loop(0, NCH, rbody, 0)

        @pl.when(last)
        def _():
            def xbody(j, carry):
                for m in range(CHUNK // HID):
                    dxidx[j, pl.ds(m * HID, HID)] = (
                        dxidx1[pl.ds(j * CHUNK + m * HID, HID)])
                return carry

            lax.fori_loop(0, XCH, xbody, 0)

        plsc.subcore_barrier()

        def body(g, carry):
            for b in range(NBUF):
                pltpu.async_copy(ones_v, acc.at[didx.at[g * NBUF + b]],
                                 ssem[b], add=True)
            for b in range(NBUF):
                pltpu.make_async_copy(
                    ones_v, acc.at[didx.at[g * NBUF + b]], ssem[b]).wait()
            return carry

        lax.fori_loop(0, NG, body, 0)

        @pl.when(last)
        def _():
            for t in range(XCH):
                pltpu.sync_copy(ones_v, acc.at[dxidx.at[t]], add=True)

        plsc.subcore_barrier()
        pltpu.sync_copy(acc.at[pl.ds(r0, RPT)], dv)

        def repl(g, carry):
            v16 = dv[pl.ds(g * HID, HID)]
            for t in range(HID):
                rep[g * HID + t] = jnp.full((HID,), v16[t], jnp.float32)
            return carry

        lax.fori_loop(0, RPT // HID, repl, 0)
        pltpu.sync_copy(rep, out_hbm.at[cid, pl.ds(r0, RPT)])

    return k(ei_raw)


def _sc_aggregate(table, ei3, zeros):
    @functools.partial(
        pl.kernel,
        out_type=jax.ShapeDtypeStruct((NC, NPAD, HID), jnp.float32),
        mesh=_sc_mesh(),
        compiler_params=pltpu.CompilerParams(use_tc_tiling_on_sc=False),
        scratch_types=[
            pltpu.VMEM((NCH, CHUNK), jnp.int32),
            pltpu.VMEM((NCH, CHUNK), jnp.int32),
            pltpu.VMEM((XCH, CHUNK), jnp.int32),
            pltpu.VMEM((XCH, CHUNK), jnp.int32),
            pltpu.VMEM_SHARED((NPAD, HID), jnp.float32),
            pltpu.VMEM_SHARED((NPAD, HID), jnp.float32),
        ]
        + [pltpu.VMEM((CHUNK, HID), jnp.float32)] * NBUF
        + [pltpu.SemaphoreType.DMA] * (2 * NBUF),
    )
    def k(table_hbm, ei_hbm, zeros_hbm, out_hbm,
          sidx, didx, sxidx, dxidx, acc, tsp, *rest):
        bufs = rest[:NBUF]
        gsem = rest[NBUF:2 * NBUF]
        ssem = rest[2 * NBUF:]
        cid = lax.axis_index("c")
        sid = lax.axis_index("s")
        wid = cid * NS + sid
        last = wid == NW - 1
        r0 = sid * RPT
        pltpu.sync_copy(zeros_hbm.at[pl.ds(r0, RPT)], acc.at[pl.ds(r0, RPT)])
        pltpu.sync_copy(table_hbm.at[pl.ds(r0, RPT)], tsp.at[pl.ds(r0, RPT)])
        pltpu.sync_copy(ei_hbm.at[0, pl.ds(wid * NCH, NCH)], sidx)
        pltpu.sync_copy(ei_hbm.at[1, pl.ds(wid * NCH, NCH)], didx)

        @pl.when(last)
        def _():
            pltpu.sync_copy(ei_hbm.at[0, pl.ds(NW * NCH, XCH)], sxidx)
            pltpu.sync_copy(ei_hbm.at[1, pl.ds(NW * NCH, XCH)], dxidx)

        plsc.subcore_barrier()

        for b in range(NBUF):
            pltpu.async_copy(tsp.at[sidx.at[b]], bufs[b], gsem[b])

        def body(g, carry):
            for b in range(NBUF):
                j = g * NBUF + b
                pltpu.make_async_copy(tsp.at[sidx.at[j]], bufs[b],
                                      gsem[b]).wait()
                pltpu.async_copy(bufs[b], acc.at[didx.at[j]], ssem[b],
                                 add=True)
            for b in range(NBUF):
                j = g * NBUF + b
                pltpu.make_async_copy(bufs[b], acc.at[didx.at[j]],
                                      ssem[b]).wait()

                @pl.when(g + 1 < NG)
                def _():
                    pltpu.async_copy(tsp.at[sidx.at[j + NBUF]], bufs[b],
                                     gsem[b])

            return carry

        lax.fori_loop(0, NG, body, 0)

        @pl.when(last)
        def _():
            for t in range(XCH):
                pltpu.async_copy(tsp.at[sxidx.at[t]], bufs[0],
                                 gsem[0]).wait()
                pltpu.sync_copy(bufs[0], acc.at[dxidx.at[t]], add=True)

        plsc.subcore_barrier()
        pltpu.sync_copy(acc.at[pl.ds(r0, RPT)],
                        out_hbm.at[cid, pl.ds(r0, RPT)])

    return k(table, ei3, zeros)


def _tc_matmul(xr, w1):
    def body(xr_ref, w1_ref, z1f_ref):
        parts = [
            jnp.dot(xr_ref[:, k, :], w1_ref[...],
                    preferred_element_type=jnp.float32)
            for k in range(8)
        ]
        z1f_ref[pl.ds(0, NFV)] = jnp.concatenate(parts, axis=1)
        z1f_ref[pl.ds(NFV, NF - NFV)] = jnp.zeros((NF - NFV, 128),
                                                  jnp.float32)

    return pl.pallas_call(
        body,
        out_shape=jax.ShapeDtypeStruct((NF, 128), jnp.float32),
    )(xr, w1)


def _tc_scale(z1f, degpf):
    def body(z1f_ref, degpf_ref, zt1f_ref, dinvf_ref):
        deg = degpf_ref[0] + degpf_ref[1] + 1.0
        dinv = lax.rsqrt(deg)
        zt1f_ref[...] = dinv * z1f_ref[...]
        dinvf_ref[...] = dinv

    return pl.pallas_call(
        body,
        out_shape=(
            jax.ShapeDtypeStruct((NF, 128), jnp.float32),
            jax.ShapeDtypeStruct((NF, 128), jnp.float32),
        ),
    )(z1f, degpf)


def _tc_mid(aggpf, zt1f, dinvf, b1f):
    def body(ap_ref, zt1f_ref, dinvf_ref, b1f_ref, zt2f_ref):
        dinv = dinvf_ref[...]
        pre = dinv * (ap_ref[0] + ap_ref[1] + zt1f_ref[...]) + b1f_ref[...]
        zt2f_ref[...] = dinv * jnp.maximum(pre, 0.0)

    return pl.pallas_call(
        body,
        out_shape=jax.ShapeDtypeStruct((NF, 128), jnp.float32),
    )(aggpf, zt1f, dinvf, b1f)


def _tc_post(aggpf, zt2f, dinvf, w2, b2):
    def body(ap_ref, zt2f_ref, dinvf_ref, w2_ref, b2_ref, out_ref):
        g = dinvf_ref[...] * (ap_ref[0] + ap_ref[1] + zt2f_ref[...])
        gv = g[:NFV]
        for k in range(8):
            out_ref[:, k, :] = jnp.dot(
                gv[:, 16 * k:16 * (k + 1)], w2_ref[...],
                preferred_element_type=jnp.float32) + b2_ref[...]

    return pl.pallas_call(
        body,
        out_shape=jax.ShapeDtypeStruct((NFV, 8, 128), jnp.float32),
    )(aggpf, zt2f, dinvf, w2, b2)


def kernel(x, edge_index, W1, b1, W2, b2):
    ei = edge_index.astype(jnp.int32)
    ei3 = ei.reshape(2, ECH, CHUNK)
    xr = x.reshape(NFV, 8, 128)

    zeros = jnp.zeros((NPAD, HID), jnp.float32)
    b1f = jnp.tile(b1, 8).reshape(1, 128)

    degp = _sc_degree(ei)
    z1f = _tc_matmul(xr, W1)
    zt1f, dinvf = _tc_scale(z1f, degp.reshape(NC, NF, 128))
    agg1 = _sc_aggregate(zt1f.reshape(NPAD, HID), ei3, zeros)
    zt2f = _tc_mid(agg1.reshape(NC, NF, 128), zt1f, dinvf, b1f)
    agg2 = _sc_aggregate(zt2f.reshape(NPAD, HID), ei3, zeros)
    out3 = _tc_post(agg2.reshape(NC, NF, 128), zt2f, dinvf, W2,
                    b2.reshape(1, OUT_CH))
    return out3.reshape(N, OUT_CH)
